# Initial kernel scaffold; baseline (speedup 1.0000x reference)
import jax
import jax.numpy as jnp
from jax.experimental import pallas as pl


def kernel(h, edge_index, coord, node_mask, edge_mask, edge_attr, We1, be1, We2, be2, Wn1, bn1, Wn2, bn2):
    raise NotImplementedError("write your pallas kernel here")


# speedup vs baseline: 4.2724x; 4.2724x over previous
import functools

import jax
import jax.numpy as jnp
from jax import lax
from jax.experimental import pallas as pl
from jax.experimental.pallas import tpu as pltpu
from jax.experimental.pallas import tpu_sc as plsc

N = 10000
E = 320000
D = 128
NC, NS = 2, 16
NW = NC * NS
EPW = E // NW
IB = 80
NJ = EPW // IB
BN = 2000
BE = 2000

_f32 = jnp.float32


def _p0_body(h_ref, cp_ref, ws_ref, wt_ref, be1_ref, wr_ref,
             a_ref, b_ref, c3_ref):
    h = h_ref[...]
    cp = cp_ref[...]
    sq = jnp.sum(cp * cp, axis=1, keepdims=True)
    swr = sq * wr_ref[...]
    a_ref[...] = jnp.dot(h, ws_ref[...], preferred_element_type=_f32) \
        + be1_ref[...] + swr
    b_ref[...] = jnp.dot(h, wt_ref[...], preferred_element_type=_f32) + swr
    c3_ref[...] = jnp.concatenate(
        [cp[:, :3], jnp.zeros((BN, D - 3), _f32)], axis=1)


def _p0(h, cp, ws, wt, be1, wr):
    grid = (N // BN,)
    return pl.pallas_call(
        _p0_body,
        grid=grid,
        in_specs=[
            pl.BlockSpec((BN, D), lambda i: (i, 0)),
            pl.BlockSpec((BN, 8), lambda i: (i, 0)),
            pl.BlockSpec((D, D), lambda i: (0, 0)),
            pl.BlockSpec((D, D), lambda i: (0, 0)),
            pl.BlockSpec((1, D), lambda i: (0, 0)),
            pl.BlockSpec((1, D), lambda i: (0, 0)),
        ],
        out_specs=[
            pl.BlockSpec((BN, D), lambda i: (i, 0)),
            pl.BlockSpec((BN, D), lambda i: (i, 0)),
            pl.BlockSpec((BN, D), lambda i: (i, 0)),
        ],
        out_shape=[
            jax.ShapeDtypeStruct((N, D), _f32),
            jax.ShapeDtypeStruct((N, D), _f32),
            jax.ShapeDtypeStruct((N, D), _f32),
        ],
    )(h, cp, ws, wt, be1, wr)


def _p1_body(a_hbm, b_hbm, c3_hbm, wr2_hbm, row_hbm, col_hbm, g_hbm,
             idxr_v, idxc_v, bufr_v, bufc_v, cbufr_v, cbufc_v, wr_v,
             semr, semc, semcr, semcc):
    w = lax.axis_index("s") * NC + lax.axis_index("c")
    pltpu.sync_copy(row_hbm.at[w], idxr_v)
    pltpu.sync_copy(col_hbm.at[w], idxc_v)
    pltpu.sync_copy(wr2_hbm, wr_v)
    base = w * EPW
    wrv = [wr_v[pl.ds(q * 16, 16)] for q in range(8)]

    def chunk(j, _):
        cr = pltpu.async_copy(a_hbm.at[idxr_v.at[j]], bufr_v, semr)
        cc = pltpu.async_copy(b_hbm.at[idxc_v.at[j]], bufc_v, semc)
        ccr = pltpu.async_copy(c3_hbm.at[idxr_v.at[j]], cbufr_v, semcr)
        ccc = pltpu.async_copy(c3_hbm.at[idxc_v.at[j]], cbufc_v, semcc)
        cr.wait()
        cc.wait()
        ccr.wait()
        ccc.wait()

        def edge(e, _):
            p = cbufr_v[e, pl.ds(0, 16)] * cbufc_v[e, pl.ds(0, 16)]
            dot3 = p[0] + p[1] + p[2]
            for q in range(8):
                sl = pl.ds(q * 16, 16)
                bufr_v[e, sl] = (bufr_v[e, sl] + bufc_v[e, sl]
                                 + dot3 * wrv[q])
            return 0

        lax.fori_loop(0, IB, edge, 0)
        pltpu.sync_copy(bufr_v, g_hbm.at[pl.ds(base + j * IB, IB)])
        return 0

    lax.fori_loop(0, NJ, chunk, 0)


def _p1(a, b, c3, wr2, row3d, col3d):
    mesh = plsc.VectorSubcoreMesh(core_axis_name="c", subcore_axis_name="s",
                                  num_cores=NC, num_subcores=NS)
    f = functools.partial(
        pl.kernel, _p1_body, mesh=mesh,
        out_type=jax.ShapeDtypeStruct((E, D), _f32),
        scratch_types=[
            pltpu.VMEM((NJ, IB), jnp.int32),
            pltpu.VMEM((NJ, IB), jnp.int32),
            pltpu.VMEM((IB, D), _f32),
            pltpu.VMEM((IB, D), _f32),
            pltpu.VMEM((IB, D), _f32),
            pltpu.VMEM((IB, D), _f32),
            pltpu.VMEM((D,), _f32),
            pltpu.SemaphoreType.DMA,
            pltpu.SemaphoreType.DMA,
            pltpu.SemaphoreType.DMA,
            pltpu.SemaphoreType.DMA,
        ],
    )()
    return f(a, b, c3, wr2, row3d, col3d)


def _p2_body(g_ref, ea_ref, wea_ref, we2_ref, be2_ref, y_ref):
    g = g_ref[...]
    e8 = ea_ref[...]
    z = g + jnp.dot(e8, wea_ref[...], preferred_element_type=_f32)
    z = jnp.maximum(z, 0.0)
    y = jnp.dot(z, we2_ref[...], preferred_element_type=_f32) + be2_ref[...]
    y_ref[...] = jnp.maximum(y, 0.0) * e8[:, 4:5]


def _p2(g, ea8, wea8, we2, be2):
    grid = (E // BE,)
    return pl.pallas_call(
        _p2_body,
        grid=grid,
        in_specs=[
            pl.BlockSpec((BE, D), lambda i: (i, 0)),
            pl.BlockSpec((BE, 8), lambda i: (i, 0)),
            pl.BlockSpec((8, D), lambda i: (0, 0)),
            pl.BlockSpec((D, D), lambda i: (0, 0)),
            pl.BlockSpec((1, D), lambda i: (0, 0)),
        ],
        out_specs=pl.BlockSpec((BE, D), lambda i: (i, 0)),
        out_shape=jax.ShapeDtypeStruct((E, D), _f32),
    )(g, ea8, wea8, we2, be2)


_SP = 632
_NP = NS * _SP


def _p3_body(y_hbm, row_hbm, agg_hbm, agg_sh, idx_v, ybuf_v, zbuf_v):
    c = lax.axis_index("c")
    s = lax.axis_index("s")
    w = s * NC + c

    def zrow(r, _):
        for q in range(8):
            zbuf_v[r, pl.ds(q * 16, 16)] = jnp.zeros((16,), _f32)
        return 0

    lax.fori_loop(0, 8, zrow, 0)

    def zcopy(k, _):
        pltpu.sync_copy(zbuf_v, agg_sh.at[pl.ds(s * _SP + k * 8, 8)])
        return 0

    lax.fori_loop(0, _SP // 8, zcopy, 0)
    plsc.subcore_barrier()

    pltpu.sync_copy(row_hbm.at[w], idx_v)
    base = w * EPW

    def chunk(j, _):
        pltpu.sync_copy(y_hbm.at[pl.ds(base + j * IB, IB)], ybuf_v)
        pltpu.sync_copy(ybuf_v, agg_sh.at[idx_v.at[j]], add=True)
        return 0

    lax.fori_loop(0, NJ, chunk, 0)
    plsc.subcore_barrier()

    pltpu.sync_copy(agg_sh.at[pl.ds(s * _SP, _SP)],
                    agg_hbm.at[pl.ds(c * _NP + s * _SP, _SP)])


def _p3(y, row3d):
    mesh = plsc.VectorSubcoreMesh(core_axis_name="c", subcore_axis_name="s",
                                  num_cores=NC, num_subcores=NS)
    f = functools.partial(
        pl.kernel, _p3_body, mesh=mesh,
        out_type=jax.ShapeDtypeStruct((NC * _NP, D), _f32),
        scratch_types=[
            pltpu.VMEM_SHARED((_NP, D), _f32),
            pltpu.VMEM((NJ, IB), jnp.int32),
            pltpu.VMEM((IB, D), _f32),
            pltpu.VMEM((8, D), _f32),
        ],
    )()
    return f(y, row3d)


def _p4_body(h_ref, a0_ref, a1_ref, wh_ref, wa_ref, bn1_ref, wn2_ref,
             bn2_ref, out_ref):
    h = h_ref[...]
    a = a0_ref[...] + a1_ref[...]
    t = jnp.dot(h, wh_ref[...], preferred_element_type=_f32)
    t += jnp.dot(a, wa_ref[...], preferred_element_type=_f32)
    t = jnp.maximum(t + bn1_ref[...], 0.0)
    out_ref[...] = h + jnp.dot(t, wn2_ref[...],
                               preferred_element_type=_f32) + bn2_ref[...]


def _p4(h, a0, a1, wh, wa, bn1, wn2, bn2):
    grid = (N // BN,)
    return pl.pallas_call(
        _p4_body,
        grid=grid,
        in_specs=[
            pl.BlockSpec((BN, D), lambda i: (i, 0)),
            pl.BlockSpec((BN, D), lambda i: (i, 0)),
            pl.BlockSpec((BN, D), lambda i: (i, 0)),
            pl.BlockSpec((D, D), lambda i: (0, 0)),
            pl.BlockSpec((D, D), lambda i: (0, 0)),
            pl.BlockSpec((1, D), lambda i: (0, 0)),
            pl.BlockSpec((D, D), lambda i: (0, 0)),
            pl.BlockSpec((1, D), lambda i: (0, 0)),
        ],
        out_specs=pl.BlockSpec((BN, D), lambda i: (i, 0)),
        out_shape=jax.ShapeDtypeStruct((N, D), _f32),
    )(h, a0, a1, wh, wa, bn1, wn2, bn2)


def kernel(h, edge_index, coord, node_mask, edge_mask, edge_attr,
           We1, be1, We2, be2, Wn1, bn1, Wn2, bn2):
    row = edge_index[0]
    col = edge_index[1]
    row3d = row.reshape(NW, NJ, IB)
    col3d = col.reshape(NW, NJ, IB)

    cp = jnp.concatenate([coord, jnp.zeros((N, 5), _f32)], axis=1)

    ws = We1[:D]
    wt = We1[D:2 * D]
    w_r = We1[2 * D]
    wea = We1[2 * D + 1:]

    wea8 = jnp.concatenate([wea, jnp.zeros((4, D), _f32)], axis=0)
    ea8 = jnp.concatenate(
        [edge_attr, edge_mask, jnp.zeros((E, 3), _f32)], axis=1)

    a, b, c3 = _p0(h, cp, ws, wt, be1[None, :], w_r[None, :])
    g = _p1(a, b, c3, -2.0 * w_r, row3d, col3d)
    y = _p2(g, ea8, wea8, We2, be2[None, :])
    agg2 = _p3(y, row3d)
    h_out = _p4(h, agg2[:N], agg2[_NP:_NP + N],
                Wn1[:D], Wn1[D:], bn1[None, :], Wn2, bn2[None, :])
    return (h_out, coord, edge_attr)
